# Initial kernel scaffold; baseline (speedup 1.0000x reference)
import jax
import jax.numpy as jnp
from jax.experimental import pallas as pl


def kernel(x, edge_index, edge_type, params):
    raise NotImplementedError("write your pallas kernel here")


# speedup vs baseline: 3.7462x; 3.7462x over previous
import functools

import jax
import jax.numpy as jnp
from jax import lax
from jax.experimental import pallas as pl
from jax.experimental.pallas import tpu as pltpu
from jax.experimental.pallas import tpu_sc as plsc

NN = 10000
EE = 160000
DD = 256
HH = 256
OO = 256
RR = 6
NB = 30
LL = 3

E_PAD = 163840
RN = RR * NN
RN_PAD = 61440
CK = 128
BLK = 400
NBLK = NN // BLK

_f32 = jnp.float32
_i32 = jnp.int32


def _dot(a, b):
    return jnp.dot(a, b, preferred_element_type=_f32)


def _tcw_body(att_ref, basis_ref, out_ref):
    out_ref[...] = _dot(att_ref[0], basis_ref[0])[None]


def _tc_relweights(att, basis):
    bflat = basis.reshape(LL, NB, HH * HH)
    out = pl.pallas_call(
        _tcw_body,
        grid=(LL, (HH * HH) // 2048),
        in_specs=[
            pl.BlockSpec((1, RR, NB), lambda l, c: (l, 0, 0)),
            pl.BlockSpec((1, NB, 2048), lambda l, c: (l, 0, c)),
        ],
        out_specs=pl.BlockSpec((1, RR, 2048), lambda l, c: (l, 0, c)),
        out_shape=jax.ShapeDtypeStruct((LL, RR, HH * HH), _f32),
    )(att, bflat)
    return out.reshape(LL, RR, HH, HH)


def _tcin_body(x_ref, win_ref, bin_ref, w_ref, root_ref,
               h_ref, hw_ref, hroot_ref):
    h = jnp.maximum(_dot(x_ref[...], win_ref[...]) + bin_ref[...], 0.0)
    h_ref[...] = h
    for r in range(RR):
        hw_ref[r] = _dot(h, w_ref[r])
    hroot_ref[...] = _dot(h, root_ref[...])


def _tc_input(x, w_in, b_in, w0, root0):
    return pl.pallas_call(
        _tcin_body,
        grid=(NBLK,),
        in_specs=[
            pl.BlockSpec((BLK, DD), lambda i: (i, 0)),
            pl.BlockSpec((DD, HH), lambda i: (0, 0)),
            pl.BlockSpec((1, HH), lambda i: (0, 0)),
            pl.BlockSpec((RR, HH, HH), lambda i: (0, 0, 0)),
            pl.BlockSpec((HH, HH), lambda i: (0, 0)),
        ],
        out_specs=[
            pl.BlockSpec((BLK, HH), lambda i: (i, 0)),
            pl.BlockSpec((RR, BLK, HH), lambda i: (0, i, 0)),
            pl.BlockSpec((BLK, HH), lambda i: (i, 0)),
        ],
        out_shape=[
            jax.ShapeDtypeStruct((NN, HH), _f32),
            jax.ShapeDtypeStruct((RR, NN, HH), _f32),
            jax.ShapeDtypeStruct((NN, HH), _f32),
        ],
    )(x, w_in, b_in, w0, root0)


def _tcstats_body(hroot_ref, a0_ref, a1_ref, bias_ref, z_ref, st_ref, acc_ref):
    i = pl.program_id(0)

    @pl.when(i == 0)
    def _():
        acc_ref[...] = jnp.zeros_like(acc_ref)

    z = hroot_ref[...] + jnp.concatenate([a0_ref[...], a1_ref[...]], axis=-1) \
        + bias_ref[...]
    z_ref[...] = z
    acc_ref[0:1] += jnp.sum(z, axis=0, keepdims=True)
    acc_ref[1:2] += jnp.sum(z * z, axis=0, keepdims=True)

    @pl.when(i == NBLK - 1)
    def _():
        st_ref[...] = acc_ref[...]


def _tc_stats(hroot, agg0, agg1, bias):
    return pl.pallas_call(
        _tcstats_body,
        grid=(NBLK,),
        in_specs=[
            pl.BlockSpec((BLK, HH), lambda i: (i, 0)),
            pl.BlockSpec((BLK, 128), lambda i: (i, 0)),
            pl.BlockSpec((BLK, 128), lambda i: (i, 0)),
            pl.BlockSpec((1, HH), lambda i: (0, 0)),
        ],
        out_specs=[
            pl.BlockSpec((BLK, HH), lambda i: (i, 0)),
            pl.BlockSpec((2, HH), lambda i: (0, 0)),
        ],
        out_shape=[
            jax.ShapeDtypeStruct((NN, HH), _f32),
            jax.ShapeDtypeStruct((2, HH), _f32),
        ],
        scratch_shapes=[pltpu.VMEM((2, HH), _f32)],
    )(hroot, agg0, agg1, bias)


def _bn_relu(z, st_ref, gamma_ref, beta_ref):
    inv_n = 1.0 / NN
    mu = st_ref[0:1] * inv_n
    var = st_ref[1:2] * inv_n - mu * mu
    rs = lax.rsqrt(var + 1e-5)
    return jnp.maximum((z - mu) * rs * gamma_ref[...] + beta_ref[...], 0.0)


def _tcb_body(z_ref, st_ref, gamma_ref, beta_ref, hprev_ref, w_ref, root_ref,
              h_ref, hw_ref, hroot_ref, *, residual):
    hn = _bn_relu(z_ref[...], st_ref, gamma_ref, beta_ref)
    if residual:
        hn = hn + hprev_ref[...]
    h_ref[...] = hn
    for r in range(RR):
        hw_ref[r] = _dot(hn, w_ref[r])
    hroot_ref[...] = _dot(hn, root_ref[...])


def _tc_mid(z, stats, gamma, beta, hprev, wnext, rootnext, residual):
    return pl.pallas_call(
        functools.partial(_tcb_body, residual=residual),
        grid=(NBLK,),
        in_specs=[
            pl.BlockSpec((BLK, HH), lambda i: (i, 0)),
            pl.BlockSpec((2, HH), lambda i: (0, 0)),
            pl.BlockSpec((1, HH), lambda i: (0, 0)),
            pl.BlockSpec((1, HH), lambda i: (0, 0)),
            pl.BlockSpec((BLK, HH), lambda i: (i, 0)),
            pl.BlockSpec((RR, HH, HH), lambda i: (0, 0, 0)),
            pl.BlockSpec((HH, HH), lambda i: (0, 0)),
        ],
        out_specs=[
            pl.BlockSpec((BLK, HH), lambda i: (i, 0)),
            pl.BlockSpec((RR, BLK, HH), lambda i: (0, i, 0)),
            pl.BlockSpec((BLK, HH), lambda i: (i, 0)),
        ],
        out_shape=[
            jax.ShapeDtypeStruct((NN, HH), _f32),
            jax.ShapeDtypeStruct((RR, NN, HH), _f32),
            jax.ShapeDtypeStruct((NN, HH), _f32),
        ],
    )(z, stats, gamma, beta, hprev, wnext, rootnext)


def _tcb2_body(z_ref, st_ref, gamma_ref, beta_ref, hprev_ref,
               naw1_ref, nab1_ref, naw2_ref, nab2_ref, w1a_ref, w1b_ref,
               emb_ref, p_ref, q_ref, na_ref):
    hn = _bn_relu(z_ref[...], st_ref, gamma_ref, beta_ref) + hprev_ref[...]
    emb_ref[...] = hn
    na1 = jnp.maximum(_dot(hn, naw1_ref[...]) + nab1_ref[...], 0.0)
    na_ref[...] = _dot(na1, naw2_ref[...]) + nab2_ref[...]
    p_ref[...] = _dot(hn, w1a_ref[...])
    q_ref[...] = _dot(hn, w1b_ref[...])


def _tc_last(z, stats, gamma, beta, hprev, na_w1, na_b1, na_w2, na_b2,
             w1a, w1b):
    return pl.pallas_call(
        _tcb2_body,
        grid=(NBLK,),
        in_specs=[
            pl.BlockSpec((BLK, HH), lambda i: (i, 0)),
            pl.BlockSpec((2, HH), lambda i: (0, 0)),
            pl.BlockSpec((1, HH), lambda i: (0, 0)),
            pl.BlockSpec((1, HH), lambda i: (0, 0)),
            pl.BlockSpec((BLK, HH), lambda i: (i, 0)),
            pl.BlockSpec((HH, HH // 2), lambda i: (0, 0)),
            pl.BlockSpec((1, HH // 2), lambda i: (0, 0)),
            pl.BlockSpec((HH // 2, 1), lambda i: (0, 0)),
            pl.BlockSpec((1, 1), lambda i: (0, 0)),
            pl.BlockSpec((HH, HH // 2), lambda i: (0, 0)),
            pl.BlockSpec((HH, HH // 2), lambda i: (0, 0)),
        ],
        out_specs=[
            pl.BlockSpec((BLK, HH), lambda i: (i, 0)),
            pl.BlockSpec((BLK, HH // 2), lambda i: (i, 0)),
            pl.BlockSpec((BLK, HH // 2), lambda i: (i, 0)),
            pl.BlockSpec((BLK, 1), lambda i: (i, 0)),
        ],
        out_shape=[
            jax.ShapeDtypeStruct((NN, HH), _f32),
            jax.ShapeDtypeStruct((NN, HH // 2), _f32),
            jax.ShapeDtypeStruct((NN, HH // 2), _f32),
            jax.ShapeDtypeStruct((NN, 1), _f32),
        ],
    )(z, stats, gamma, beta, hprev, na_w1, na_b1, na_w2, na_b2, w1a, w1b)


def _tcwiden_body(r_ref, o_ref):
    o_ref[...] = jnp.broadcast_to(r_ref[...], (8192, 16))


def _tc_widen(recip):
    return pl.pallas_call(
        _tcwiden_body,
        grid=(E_PAD // 8192,),
        in_specs=[pl.BlockSpec((8192, 1), lambda i: (i, 0))],
        out_specs=pl.BlockSpec((8192, 16), lambda i: (i, 0)),
        out_shape=jax.ShapeDtypeStruct((E_PAD, 16), _f32),
    )(recip.reshape(E_PAD, 1))


def _tcsoft_body(na_ref, w1c_ref, b1_ref, naw_ref, w1cb_ref):
    nav = na_ref[...]
    e = jnp.exp(nav - jnp.max(nav))
    naw_ref[...] = e / jnp.sum(e)
    w1cb_ref[...] = w1c_ref[...] + b1_ref[...]


def _tc_softmax(na, w1c, b1):
    return pl.pallas_call(
        _tcsoft_body,
        out_shape=[
            jax.ShapeDtypeStruct((NN, 1), _f32),
            jax.ShapeDtypeStruct((RR, HH // 2), _f32),
        ],
    )(na, w1c, b1)


def _tcg_body(naw_ref, emb_ref, wout_ref, bout_ref, g_ref, acc_ref):
    i = pl.program_id(0)

    @pl.when(i == 0)
    def _():
        acc_ref[...] = jnp.zeros_like(acc_ref)

    acc_ref[...] += lax.dot_general(
        naw_ref[...], emb_ref[...], (((0,), (0,)), ((), ())),
        preferred_element_type=_f32)

    @pl.when(i == NBLK - 1)
    def _():
        g_ref[...] = _dot(acc_ref[...], wout_ref[...]) + bout_ref[...]


def _tc_graph_out(naw, emb, w_out, b_out):
    return pl.pallas_call(
        _tcg_body,
        grid=(NBLK,),
        in_specs=[
            pl.BlockSpec((BLK, 1), lambda i: (i, 0)),
            pl.BlockSpec((BLK, HH), lambda i: (i, 0)),
            pl.BlockSpec((HH, OO), lambda i: (0, 0)),
            pl.BlockSpec((1, OO), lambda i: (0, 0)),
        ],
        out_specs=pl.BlockSpec((1, OO), lambda i: (0, 0)),
        out_shape=jax.ShapeDtypeStruct((1, OO), _f32),
        scratch_shapes=[pltpu.VMEM((1, OO), _f32)],
    )(naw, emb, w_out, b_out)


_MESH = plsc.VectorSubcoreMesh(core_axis_name="c", subcore_axis_name="s")


def _sc_prep_body(idxrn_hbm, maske_hbm, recip_hbm,
                  cnt_sh, idx_v, val_v, out_v, sem):
    c = lax.axis_index("c")
    s = lax.axis_index("s")

    @pl.when(c == 0)
    def _():
        for k in range(8):
            val_v[pl.ds(k * 16, 16)] = jnp.zeros((16,), _f32)

        @pl.loop(0, 30)
        def _(j):
            pltpu.sync_copy(val_v, cnt_sh.at[pl.ds(s * 3840 + j * 128, 128)])

    plsc.subcore_barrier()

    @pl.when(c == 0)
    def _():
        @pl.loop(0, 80)
        def _(t):
            off = s * 10240 + t * CK
            pltpu.sync_copy(idxrn_hbm.at[pl.ds(off, CK)], idx_v)
            pltpu.sync_copy(maske_hbm.at[pl.ds(off, CK)], val_v)
            pltpu.sync_copy(val_v, cnt_sh.at[idx_v], add=True)

    plsc.subcore_barrier()

    @pl.when(c == 0)
    def _():
        @pl.loop(0, 30)
        def _(j):
            sl = pl.ds(s * 3840 + j * 128, 128)
            pltpu.sync_copy(cnt_sh.at[sl], out_v)
            for k in range(8):
                v = out_v[pl.ds(k * 16, 16)]
                out_v[pl.ds(k * 16, 16)] = 1.0 / jnp.maximum(v, 1.0)
            pltpu.sync_copy(out_v, cnt_sh.at[sl])

    plsc.subcore_barrier()

    @pl.when(c == 0)
    def _():
        @pl.loop(0, 80)
        def _(t):
            off = s * 10240 + t * CK
            pltpu.sync_copy(idxrn_hbm.at[pl.ds(off, CK)], idx_v)
            pltpu.async_copy(cnt_sh.at[idx_v], out_v, sem).wait()
            pltpu.sync_copy(maske_hbm.at[pl.ds(off, CK)], val_v)
            for k in range(8):
                sl = pl.ds(k * 16, 16)
                out_v[sl] = out_v[sl] * val_v[sl]
            pltpu.sync_copy(out_v, recip_hbm.at[pl.ds(off, CK)])


_sc_prep = functools.partial(
    pl.kernel,
    _sc_prep_body,
    out_type=jax.ShapeDtypeStruct((E_PAD,), _f32),
    mesh=_MESH,
    scratch_types=[
        pltpu.VMEM_SHARED((RN_PAD,), _f32),
        pltpu.VMEM((CK,), _i32),
        pltpu.VMEM((CK,), _f32),
        pltpu.VMEM((CK,), _f32),
        pltpu.SemaphoreType.DMA,
    ],
)()


def _sc_agg_body(hw2_hbm, base2_hbm, dst_hbm, recip2_hbm,
                 agg0_hbm, agg1_hbm,
                 acc_sh, rows_v, bidx_v, didx_v, rrep_v, sem):
    c = lax.axis_index("c")
    s = lax.axis_index("s")

    @pl.loop(0, CK)
    def _(j):
        for k in range(8):
            rows_v[j, pl.ds(k * 16, 16)] = jnp.zeros((16,), _f32)

    @pl.loop(0, 5)
    def _(j):
        pltpu.sync_copy(rows_v, acc_sh.at[pl.ds(s * 640 + j * CK, CK)])

    plsc.subcore_barrier()

    @pl.loop(0, 80)
    def _(t):
        off = s * 10240 + t * CK
        pltpu.sync_copy(base2_hbm.at[pl.ds(off, CK)], bidx_v)
        for k in range(8):
            sl = pl.ds(k * 16, 16)
            bidx_v[sl] = bidx_v[sl] + c
        cp = pltpu.async_copy(hw2_hbm.at[bidx_v], rows_v, sem)
        pltpu.sync_copy(dst_hbm.at[pl.ds(off, CK)], didx_v)
        pltpu.sync_copy(recip2_hbm.at[pl.ds(off, CK)], rrep_v)
        cp.wait()

        @pl.loop(0, CK)
        def _(j):
            rv = rrep_v[j]
            for k in range(8):
                sl = pl.ds(k * 16, 16)
                rows_v[j, sl] = rows_v[j, sl] * rv

        pltpu.sync_copy(rows_v, acc_sh.at[didx_v], add=True)

    plsc.subcore_barrier()

    @pl.when(c == 0)
    def _():
        pltpu.sync_copy(acc_sh.at[pl.ds(s * 640, 640)],
                        agg0_hbm.at[pl.ds(s * 640, 640)])

    @pl.when(c == 1)
    def _():
        pltpu.sync_copy(acc_sh.at[pl.ds(s * 640, 640)],
                        agg1_hbm.at[pl.ds(s * 640, 640)])


_sc_agg = functools.partial(
    pl.kernel,
    _sc_agg_body,
    out_type=[
        jax.ShapeDtypeStruct((10240, 128), _f32),
        jax.ShapeDtypeStruct((10240, 128), _f32),
    ],
    mesh=_MESH,
    scratch_types=[
        pltpu.VMEM_SHARED((10240, 128), _f32),
        pltpu.VMEM((CK, 128), _f32),
        pltpu.VMEM((CK,), _i32),
        pltpu.VMEM((CK,), _i32),
        pltpu.VMEM((CK, 16), _f32),
        pltpu.SemaphoreType.DMA,
    ],
)()


def _sc_edge_body(p_hbm, q_hbm, w1cb_hbm,
                  src_hbm, dst_hbm, et_hbm, u_hbm,
                  prow_v, qrow_v, crow_v, sidx_v, didx_v, tidx_v, sem):
    c = lax.axis_index("c")
    s = lax.axis_index("s")
    wid = s * 2 + c

    @pl.loop(0, 40)
    def _(t):
        off = wid * 5120 + t * CK
        pltpu.sync_copy(src_hbm.at[pl.ds(off, CK)], sidx_v)
        cp1 = pltpu.async_copy(p_hbm.at[sidx_v], prow_v, sem)
        pltpu.sync_copy(dst_hbm.at[pl.ds(off, CK)], didx_v)
        cp2 = pltpu.async_copy(q_hbm.at[didx_v], qrow_v, sem)
        pltpu.sync_copy(et_hbm.at[pl.ds(off, CK)], tidx_v)
        cp3 = pltpu.async_copy(w1cb_hbm.at[tidx_v], crow_v, sem)
        cp1.wait()
        cp2.wait()
        cp3.wait()

        @pl.loop(0, CK)
        def _(j):
            for k in range(8):
                sl = pl.ds(k * 16, 16)
                prow_v[j, sl] = prow_v[j, sl] + qrow_v[j, sl] + crow_v[j, sl]

        pltpu.sync_copy(prow_v, u_hbm.at[pl.ds(off, CK)])


_sc_edge = functools.partial(
    pl.kernel,
    _sc_edge_body,
    out_type=jax.ShapeDtypeStruct((E_PAD, 128), _f32),
    mesh=_MESH,
    scratch_types=[
        pltpu.VMEM((CK, 128), _f32),
        pltpu.VMEM((CK, 128), _f32),
        pltpu.VMEM((CK, 128), _f32),
        pltpu.VMEM((CK,), _i32),
        pltpu.VMEM((CK,), _i32),
        pltpu.VMEM((CK,), _i32),
        pltpu.SemaphoreType.DMA,
    ],
)()


def _tcea_body(u_ref, w2_ref, b2_ref, ea_ref):
    v = _dot(jnp.maximum(u_ref[...], 0.0), w2_ref[...]) + b2_ref[...]
    ea_ref[...] = 1.0 / (1.0 + jnp.exp(-v))


def _tc_ea(u, w2, b2):
    return pl.pallas_call(
        _tcea_body,
        grid=(E_PAD // 2048,),
        in_specs=[
            pl.BlockSpec((2048, 128), lambda i: (i, 0)),
            pl.BlockSpec((128, 1), lambda i: (0, 0)),
            pl.BlockSpec((1, 1), lambda i: (0, 0)),
        ],
        out_specs=pl.BlockSpec((2048, 1), lambda i: (i, 0)),
        out_shape=jax.ShapeDtypeStruct((E_PAD, 1), _f32),
    )(u, w2, b2)


def kernel(x, edge_index, edge_type, params):
    p = params
    src = edge_index[0].astype(_i32)
    dst = edge_index[1].astype(_i32)
    et = edge_type.astype(_i32)

    pad = E_PAD - EE
    srcp = jnp.pad(src, (0, pad))
    dstp = jnp.pad(dst, (0, pad))
    etp = jnp.pad(et, (0, pad))
    base2 = jnp.pad((et * NN + src) * 2, (0, pad))
    idxrn = jnp.pad(et * NN + dst, (0, pad), constant_values=RN_PAD - 1)
    maske = jnp.pad(jnp.ones((EE,), _f32), (0, pad))

    wt = _tc_relweights(p["att"], p["basis"])

    recip = _sc_prep(idxrn, maske)
    recip2 = _tc_widen(recip)

    h, hw, hroot = _tc_input(x, p["W_in"], p["b_in"].reshape(1, HH),
                             wt[0], p["root"][0])

    emb = naw = na = pq_p = pq_q = None
    for i in range(LL):
        hw2 = hw.reshape(RR * NN * 2, 128)
        agg0, agg1 = _sc_agg(hw2, base2, dstp, recip2)
        z, stats = _tc_stats(hroot, agg0, agg1, p["conv_bias"][i].reshape(1, HH))
        gamma = p["bn_gamma"][i].reshape(1, HH)
        beta = p["bn_beta"][i].reshape(1, HH)
        if i < LL - 1:
            h, hw, hroot = _tc_mid(z, stats, gamma, beta, h,
                                   wt[i + 1], p["root"][i + 1],
                                   residual=(i > 0))
        else:
            emb, pq_p, pq_q, na = _tc_last(
                z, stats, gamma, beta, h,
                p["na_W1"], p["na_b1"].reshape(1, HH // 2),
                p["na_W2"], p["na_b2"].reshape(1, 1),
                p["ea_W1"][:HH], p["ea_W1"][HH:2 * HH])

    naw, w1cb = _tc_softmax(na, p["ea_W1"][2 * HH:], p["ea_b1"].reshape(1, HH // 2))
    g = _tc_graph_out(naw, emb, p["W_out"], p["b_out"].reshape(1, OO))

    ea_u = _sc_edge(pq_p, pq_q, w1cb, srcp, dstp, etp)
    ea_full = _tc_ea(ea_u, p["ea_W2"], p["ea_b2"].reshape(1, 1))
    ea = ea_full[:EE]

    return (g, emb, naw, ea)


# speedup vs baseline: 5.4626x; 1.4582x over previous
import functools

import jax
import jax.numpy as jnp
from jax import lax
from jax.experimental import pallas as pl
from jax.experimental.pallas import tpu as pltpu
from jax.experimental.pallas import tpu_sc as plsc

NN = 10000
EE = 160000
DD = 256
HH = 256
OO = 256
RR = 6
NB = 30
LL = 3

E_PAD = 163840
RN = RR * NN
RN_PAD = 61440
CK = 128
BLK = 400
NBLK = NN // BLK

_f32 = jnp.float32
_i32 = jnp.int32


def _dot(a, b):
    return jnp.dot(a, b, preferred_element_type=_f32)


def _tcw_body(att_ref, basis_ref, out_ref):
    out_ref[...] = _dot(att_ref[0], basis_ref[0])[None]


def _tc_relweights(att, basis):
    bflat = basis.reshape(LL, NB, HH * HH)
    out = pl.pallas_call(
        _tcw_body,
        grid=(LL, (HH * HH) // 2048),
        in_specs=[
            pl.BlockSpec((1, RR, NB), lambda l, c: (l, 0, 0)),
            pl.BlockSpec((1, NB, 2048), lambda l, c: (l, 0, c)),
        ],
        out_specs=pl.BlockSpec((1, RR, 2048), lambda l, c: (l, 0, c)),
        out_shape=jax.ShapeDtypeStruct((LL, RR, HH * HH), _f32),
    )(att, bflat)
    return out.reshape(LL, RR, HH, HH)


def _tcin_body(x_ref, win_ref, bin_ref, w_ref, root_ref,
               h_ref, hw_ref, hroot_ref):
    h = jnp.maximum(_dot(x_ref[...], win_ref[...]) + bin_ref[...], 0.0)
    h_ref[...] = h
    for r in range(RR):
        hw_ref[r] = _dot(h, w_ref[r])
    hroot_ref[...] = _dot(h, root_ref[...])


def _tc_input(x, w_in, b_in, w0, root0):
    return pl.pallas_call(
        _tcin_body,
        grid=(NBLK,),
        in_specs=[
            pl.BlockSpec((BLK, DD), lambda i: (i, 0)),
            pl.BlockSpec((DD, HH), lambda i: (0, 0)),
            pl.BlockSpec((1, HH), lambda i: (0, 0)),
            pl.BlockSpec((RR, HH, HH), lambda i: (0, 0, 0)),
            pl.BlockSpec((HH, HH), lambda i: (0, 0)),
        ],
        out_specs=[
            pl.BlockSpec((BLK, HH), lambda i: (i, 0)),
            pl.BlockSpec((RR, BLK, HH), lambda i: (0, i, 0)),
            pl.BlockSpec((BLK, HH), lambda i: (i, 0)),
        ],
        out_shape=[
            jax.ShapeDtypeStruct((NN, HH), _f32),
            jax.ShapeDtypeStruct((RR, NN, HH), _f32),
            jax.ShapeDtypeStruct((NN, HH), _f32),
        ],
    )(x, w_in, b_in, w0, root0)


def _tcstats_body(hroot_ref, a0_ref, a1_ref, bias_ref, z_ref, st_ref, acc_ref):
    i = pl.program_id(0)

    @pl.when(i == 0)
    def _():
        acc_ref[...] = jnp.zeros_like(acc_ref)

    z = hroot_ref[...] + jnp.concatenate([a0_ref[...], a1_ref[...]], axis=-1) \
        + bias_ref[...]
    z_ref[...] = z
    acc_ref[0:1] += jnp.sum(z, axis=0, keepdims=True)
    acc_ref[1:2] += jnp.sum(z * z, axis=0, keepdims=True)

    @pl.when(i == NBLK - 1)
    def _():
        st_ref[...] = acc_ref[...]


def _tc_stats(hroot, agg0, agg1, bias):
    return pl.pallas_call(
        _tcstats_body,
        grid=(NBLK,),
        in_specs=[
            pl.BlockSpec((BLK, HH), lambda i: (i, 0)),
            pl.BlockSpec((BLK, 128), lambda i: (i, 0)),
            pl.BlockSpec((BLK, 128), lambda i: (i, 0)),
            pl.BlockSpec((1, HH), lambda i: (0, 0)),
        ],
        out_specs=[
            pl.BlockSpec((BLK, HH), lambda i: (i, 0)),
            pl.BlockSpec((2, HH), lambda i: (0, 0)),
        ],
        out_shape=[
            jax.ShapeDtypeStruct((NN, HH), _f32),
            jax.ShapeDtypeStruct((2, HH), _f32),
        ],
        scratch_shapes=[pltpu.VMEM((2, HH), _f32)],
    )(hroot, agg0, agg1, bias)


def _bn_relu(z, st_ref, gamma_ref, beta_ref):
    inv_n = 1.0 / NN
    mu = st_ref[0:1] * inv_n
    var = st_ref[1:2] * inv_n - mu * mu
    rs = lax.rsqrt(var + 1e-5)
    return jnp.maximum((z - mu) * rs * gamma_ref[...] + beta_ref[...], 0.0)


def _tcb_body(z_ref, st_ref, gamma_ref, beta_ref, hprev_ref, w_ref, root_ref,
              h_ref, hw_ref, hroot_ref, *, residual):
    hn = _bn_relu(z_ref[...], st_ref, gamma_ref, beta_ref)
    if residual:
        hn = hn + hprev_ref[...]
    h_ref[...] = hn
    for r in range(RR):
        hw_ref[r] = _dot(hn, w_ref[r])
    hroot_ref[...] = _dot(hn, root_ref[...])


def _tc_mid(z, stats, gamma, beta, hprev, wnext, rootnext, residual):
    return pl.pallas_call(
        functools.partial(_tcb_body, residual=residual),
        grid=(NBLK,),
        in_specs=[
            pl.BlockSpec((BLK, HH), lambda i: (i, 0)),
            pl.BlockSpec((2, HH), lambda i: (0, 0)),
            pl.BlockSpec((1, HH), lambda i: (0, 0)),
            pl.BlockSpec((1, HH), lambda i: (0, 0)),
            pl.BlockSpec((BLK, HH), lambda i: (i, 0)),
            pl.BlockSpec((RR, HH, HH), lambda i: (0, 0, 0)),
            pl.BlockSpec((HH, HH), lambda i: (0, 0)),
        ],
        out_specs=[
            pl.BlockSpec((BLK, HH), lambda i: (i, 0)),
            pl.BlockSpec((RR, BLK, HH), lambda i: (0, i, 0)),
            pl.BlockSpec((BLK, HH), lambda i: (i, 0)),
        ],
        out_shape=[
            jax.ShapeDtypeStruct((NN, HH), _f32),
            jax.ShapeDtypeStruct((RR, NN, HH), _f32),
            jax.ShapeDtypeStruct((NN, HH), _f32),
        ],
    )(z, stats, gamma, beta, hprev, wnext, rootnext)


def _tcb2_body(z_ref, st_ref, gamma_ref, beta_ref, hprev_ref,
               naw1_ref, nab1_ref, naw2_ref, nab2_ref, w1a_ref, w1b_ref,
               emb_ref, p_ref, q_ref, na_ref):
    hn = _bn_relu(z_ref[...], st_ref, gamma_ref, beta_ref) + hprev_ref[...]
    emb_ref[...] = hn
    na1 = jnp.maximum(_dot(hn, naw1_ref[...]) + nab1_ref[...], 0.0)
    na_ref[...] = _dot(na1, naw2_ref[...]) + nab2_ref[...]
    p_ref[...] = _dot(hn, w1a_ref[...])
    q_ref[...] = _dot(hn, w1b_ref[...])


def _tc_last(z, stats, gamma, beta, hprev, na_w1, na_b1, na_w2, na_b2,
             w1a, w1b):
    return pl.pallas_call(
        _tcb2_body,
        grid=(NBLK,),
        in_specs=[
            pl.BlockSpec((BLK, HH), lambda i: (i, 0)),
            pl.BlockSpec((2, HH), lambda i: (0, 0)),
            pl.BlockSpec((1, HH), lambda i: (0, 0)),
            pl.BlockSpec((1, HH), lambda i: (0, 0)),
            pl.BlockSpec((BLK, HH), lambda i: (i, 0)),
            pl.BlockSpec((HH, HH // 2), lambda i: (0, 0)),
            pl.BlockSpec((1, HH // 2), lambda i: (0, 0)),
            pl.BlockSpec((HH // 2, 1), lambda i: (0, 0)),
            pl.BlockSpec((1, 1), lambda i: (0, 0)),
            pl.BlockSpec((HH, HH // 2), lambda i: (0, 0)),
            pl.BlockSpec((HH, HH // 2), lambda i: (0, 0)),
        ],
        out_specs=[
            pl.BlockSpec((BLK, HH), lambda i: (i, 0)),
            pl.BlockSpec((BLK, HH // 2), lambda i: (i, 0)),
            pl.BlockSpec((BLK, HH // 2), lambda i: (i, 0)),
            pl.BlockSpec((BLK, 1), lambda i: (i, 0)),
        ],
        out_shape=[
            jax.ShapeDtypeStruct((NN, HH), _f32),
            jax.ShapeDtypeStruct((NN, HH // 2), _f32),
            jax.ShapeDtypeStruct((NN, HH // 2), _f32),
            jax.ShapeDtypeStruct((NN, 1), _f32),
        ],
    )(z, stats, gamma, beta, hprev, na_w1, na_b1, na_w2, na_b2, w1a, w1b)


def _tcwiden_body(r_ref, o_ref):
    o_ref[...] = jnp.broadcast_to(r_ref[...], (8192, 16))


def _tc_widen(recip):
    return pl.pallas_call(
        _tcwiden_body,
        grid=(E_PAD // 8192,),
        in_specs=[pl.BlockSpec((8192, 1), lambda i: (i, 0))],
        out_specs=pl.BlockSpec((8192, 16), lambda i: (i, 0)),
        out_shape=jax.ShapeDtypeStruct((E_PAD, 16), _f32),
    )(recip.reshape(E_PAD, 1))


def _tcsoft_body(na_ref, w1c_ref, b1_ref, naw_ref, w1cb_ref):
    nav = na_ref[...]
    e = jnp.exp(nav - jnp.max(nav))
    naw_ref[...] = e / jnp.sum(e)
    w1cb_ref[...] = w1c_ref[...] + b1_ref[...]


def _tc_softmax(na, w1c, b1):
    return pl.pallas_call(
        _tcsoft_body,
        out_shape=[
            jax.ShapeDtypeStruct((NN, 1), _f32),
            jax.ShapeDtypeStruct((RR, HH // 2), _f32),
        ],
    )(na, w1c, b1)


def _tcg_body(naw_ref, emb_ref, wout_ref, bout_ref, g_ref, acc_ref):
    i = pl.program_id(0)

    @pl.when(i == 0)
    def _():
        acc_ref[...] = jnp.zeros_like(acc_ref)

    acc_ref[...] += lax.dot_general(
        naw_ref[...], emb_ref[...], (((0,), (0,)), ((), ())),
        preferred_element_type=_f32)

    @pl.when(i == NBLK - 1)
    def _():
        g_ref[...] = _dot(acc_ref[...], wout_ref[...]) + bout_ref[...]


def _tc_graph_out(naw, emb, w_out, b_out):
    return pl.pallas_call(
        _tcg_body,
        grid=(NBLK,),
        in_specs=[
            pl.BlockSpec((BLK, 1), lambda i: (i, 0)),
            pl.BlockSpec((BLK, HH), lambda i: (i, 0)),
            pl.BlockSpec((HH, OO), lambda i: (0, 0)),
            pl.BlockSpec((1, OO), lambda i: (0, 0)),
        ],
        out_specs=pl.BlockSpec((1, OO), lambda i: (0, 0)),
        out_shape=jax.ShapeDtypeStruct((1, OO), _f32),
        scratch_shapes=[pltpu.VMEM((1, OO), _f32)],
    )(naw, emb, w_out, b_out)


_MESH = plsc.VectorSubcoreMesh(core_axis_name="c", subcore_axis_name="s")


def _sc_prep_body(idxrn_hbm, maske_hbm, recip_hbm,
                  cnt_sh, idx_v, val_v, out_v, sem):
    c = lax.axis_index("c")
    s = lax.axis_index("s")

    @pl.when(c == 0)
    def _():
        for k in range(8):
            val_v[pl.ds(k * 16, 16)] = jnp.zeros((16,), _f32)

        @pl.loop(0, 30)
        def _(j):
            pltpu.sync_copy(val_v, cnt_sh.at[pl.ds(s * 3840 + j * 128, 128)])

    plsc.subcore_barrier()

    @pl.when(c == 0)
    def _():
        @pl.loop(0, 80)
        def _(t):
            off = s * 10240 + t * CK
            pltpu.sync_copy(idxrn_hbm.at[pl.ds(off, CK)], idx_v)
            pltpu.sync_copy(maske_hbm.at[pl.ds(off, CK)], val_v)
            pltpu.sync_copy(val_v, cnt_sh.at[idx_v], add=True)

    plsc.subcore_barrier()

    @pl.when(c == 0)
    def _():
        @pl.loop(0, 30)
        def _(j):
            sl = pl.ds(s * 3840 + j * 128, 128)
            pltpu.sync_copy(cnt_sh.at[sl], out_v)
            for k in range(8):
                v = out_v[pl.ds(k * 16, 16)]
                out_v[pl.ds(k * 16, 16)] = 1.0 / jnp.maximum(v, 1.0)
            pltpu.sync_copy(out_v, cnt_sh.at[sl])

    plsc.subcore_barrier()

    @pl.when(c == 0)
    def _():
        @pl.loop(0, 80)
        def _(t):
            off = s * 10240 + t * CK
            pltpu.sync_copy(idxrn_hbm.at[pl.ds(off, CK)], idx_v)
            pltpu.async_copy(cnt_sh.at[idx_v], out_v, sem).wait()
            pltpu.sync_copy(maske_hbm.at[pl.ds(off, CK)], val_v)
            for k in range(8):
                sl = pl.ds(k * 16, 16)
                out_v[sl] = out_v[sl] * val_v[sl]
            pltpu.sync_copy(out_v, recip_hbm.at[pl.ds(off, CK)])


_sc_prep = functools.partial(
    pl.kernel,
    _sc_prep_body,
    out_type=jax.ShapeDtypeStruct((E_PAD,), _f32),
    mesh=_MESH,
    scratch_types=[
        pltpu.VMEM_SHARED((RN_PAD,), _f32),
        pltpu.VMEM((CK,), _i32),
        pltpu.VMEM((CK,), _f32),
        pltpu.VMEM((CK,), _f32),
        pltpu.SemaphoreType.DMA,
    ],
)()


def _sc_agg_body(hw2_hbm, base2_hbm, dst_hbm, recip2_hbm,
                 agg0_hbm, agg1_hbm,
                 acc_sh, rows0_v, rows1_v, bidx0_v, bidx1_v,
                 didx_v, rrep_v, sem0, sem1):
    c = lax.axis_index("c")
    s = lax.axis_index("s")

    @pl.loop(0, CK)
    def _(j):
        for k in range(8):
            rows0_v[j, pl.ds(k * 16, 16)] = jnp.zeros((16,), _f32)

    for o, n in ((0, 128), (128, 128), (256, 128), (384, 128), (512, 120)):
        pltpu.sync_copy(rows0_v.at[pl.ds(0, n)],
                        acc_sh.at[pl.ds(s * 632 + o, n)])

    plsc.subcore_barrier()

    def _issue(off, bidx_v, rows_v, sem):
        pltpu.sync_copy(base2_hbm.at[pl.ds(off, CK)], bidx_v)
        for k in range(8):
            sl = pl.ds(k * 16, 16)
            bidx_v[sl] = bidx_v[sl] + c
        return pltpu.async_copy(hw2_hbm.at[bidx_v], rows_v, sem)

    def _process(off, rows_v, cp):
        pltpu.sync_copy(dst_hbm.at[pl.ds(off, CK)], didx_v)
        pltpu.sync_copy(recip2_hbm.at[pl.ds(off, CK)], rrep_v)
        cp.wait()

        @pl.loop(0, CK)
        def _(j):
            rv = rrep_v[j]
            for k in range(8):
                sl = pl.ds(k * 16, 16)
                rows_v[j, sl] = rows_v[j, sl] * rv

        pltpu.sync_copy(rows_v, acc_sh.at[didx_v], add=True)

    @pl.loop(0, 40)
    def _(g):
        off0 = s * 10240 + g * (2 * CK)
        off1 = off0 + CK
        cp0 = _issue(off0, bidx0_v, rows0_v, sem0)
        cp1 = _issue(off1, bidx1_v, rows1_v, sem1)
        _process(off0, rows0_v, cp0)
        _process(off1, rows1_v, cp1)

    plsc.subcore_barrier()

    @pl.when(c == 0)
    def _():
        pltpu.sync_copy(acc_sh.at[pl.ds(s * 632, 632)],
                        agg0_hbm.at[pl.ds(s * 632, 632)])

    @pl.when(c == 1)
    def _():
        pltpu.sync_copy(acc_sh.at[pl.ds(s * 632, 632)],
                        agg1_hbm.at[pl.ds(s * 632, 632)])


_sc_agg = functools.partial(
    pl.kernel,
    _sc_agg_body,
    out_type=[
        jax.ShapeDtypeStruct((10112, 128), _f32),
        jax.ShapeDtypeStruct((10112, 128), _f32),
    ],
    mesh=_MESH,
    scratch_types=[
        pltpu.VMEM_SHARED((10112, 128), _f32),
        pltpu.VMEM((CK, 128), _f32),
        pltpu.VMEM((CK, 128), _f32),
        pltpu.VMEM((CK,), _i32),
        pltpu.VMEM((CK,), _i32),
        pltpu.VMEM((CK,), _i32),
        pltpu.VMEM((CK, 16), _f32),
        pltpu.SemaphoreType.DMA,
        pltpu.SemaphoreType.DMA,
    ],
)()


def _sc_edge_body(p_hbm, q_hbm, src_hbm, dst_hbm, u_hbm,
                  prow0_v, prow1_v, qrow0_v, qrow1_v,
                  sidx0_v, sidx1_v, didx0_v, didx1_v, sem0, sem1):
    c = lax.axis_index("c")
    s = lax.axis_index("s")
    wid = s * 2 + c

    def _issue(off, sidx_v, didx_v, prow_v, qrow_v, sem):
        pltpu.sync_copy(src_hbm.at[pl.ds(off, CK)], sidx_v)
        cpp = pltpu.async_copy(p_hbm.at[sidx_v], prow_v, sem)
        pltpu.sync_copy(dst_hbm.at[pl.ds(off, CK)], didx_v)
        cpq = pltpu.async_copy(q_hbm.at[didx_v], qrow_v, sem)
        return cpp, cpq

    def _process(off, prow_v, qrow_v, cps):
        cps[0].wait()
        cps[1].wait()

        @pl.loop(0, CK)
        def _(j):
            for k in range(8):
                sl = pl.ds(k * 16, 16)
                prow_v[j, sl] = prow_v[j, sl] + qrow_v[j, sl]

        pltpu.sync_copy(prow_v, u_hbm.at[pl.ds(off, CK)])

    @pl.loop(0, 20)
    def _(t):
        off0 = wid * 5120 + t * (2 * CK)
        off1 = off0 + CK
        cp0 = _issue(off0, sidx0_v, didx0_v, prow0_v, qrow0_v, sem0)
        cp1 = _issue(off1, sidx1_v, didx1_v, prow1_v, qrow1_v, sem1)
        _process(off0, prow0_v, qrow0_v, cp0)
        _process(off1, prow1_v, qrow1_v, cp1)


_sc_edge = functools.partial(
    pl.kernel,
    _sc_edge_body,
    out_type=jax.ShapeDtypeStruct((E_PAD, 128), _f32),
    mesh=_MESH,
    scratch_types=[
        pltpu.VMEM((CK, 128), _f32),
        pltpu.VMEM((CK, 128), _f32),
        pltpu.VMEM((CK, 128), _f32),
        pltpu.VMEM((CK, 128), _f32),
        pltpu.VMEM((CK,), _i32),
        pltpu.VMEM((CK,), _i32),
        pltpu.VMEM((CK,), _i32),
        pltpu.VMEM((CK,), _i32),
        pltpu.SemaphoreType.DMA,
        pltpu.SemaphoreType.DMA,
    ],
)()


def _tcea_body(u_ref, et_ref, w1cb_ref, w2_ref, b2_ref, ea_ref):
    oh = (et_ref[...] == lax.broadcasted_iota(_i32, (1, 8), 1)).astype(_f32)
    v = jnp.maximum(u_ref[...] + _dot(oh, w1cb_ref[...]), 0.0)
    v = _dot(v, w2_ref[...]) + b2_ref[...]
    ea_ref[...] = 1.0 / (1.0 + jnp.exp(-v))


def _tc_ea(u, et2, w1cb8, w2, b2):
    return pl.pallas_call(
        _tcea_body,
        grid=(E_PAD // 2048,),
        in_specs=[
            pl.BlockSpec((2048, 128), lambda i: (i, 0)),
            pl.BlockSpec((2048, 1), lambda i: (i, 0)),
            pl.BlockSpec((8, 128), lambda i: (0, 0)),
            pl.BlockSpec((128, 1), lambda i: (0, 0)),
            pl.BlockSpec((1, 1), lambda i: (0, 0)),
        ],
        out_specs=pl.BlockSpec((2048, 1), lambda i: (i, 0)),
        out_shape=jax.ShapeDtypeStruct((E_PAD, 1), _f32),
    )(u, et2, w1cb8, w2, b2)


def kernel(x, edge_index, edge_type, params):
    p = params
    src = edge_index[0].astype(_i32)
    dst = edge_index[1].astype(_i32)
    et = edge_type.astype(_i32)

    pad = E_PAD - EE
    srcp = jnp.pad(src, (0, pad))
    dstp = jnp.pad(dst, (0, pad))
    etp = jnp.pad(et, (0, pad))
    base2 = jnp.pad((et * NN + src) * 2, (0, pad))
    idxrn = jnp.pad(et * NN + dst, (0, pad), constant_values=RN_PAD - 1)
    maske = jnp.pad(jnp.ones((EE,), _f32), (0, pad))

    wt = _tc_relweights(p["att"], p["basis"])

    recip = _sc_prep(idxrn, maske)
    recip2 = _tc_widen(recip)

    h, hw, hroot = _tc_input(x, p["W_in"], p["b_in"].reshape(1, HH),
                             wt[0], p["root"][0])

    emb = naw = na = pq_p = pq_q = None
    for i in range(LL):
        hw2 = hw.reshape(RR * NN * 2, 128)
        agg0, agg1 = _sc_agg(hw2, base2, dstp, recip2)
        z, stats = _tc_stats(hroot, agg0, agg1, p["conv_bias"][i].reshape(1, HH))
        gamma = p["bn_gamma"][i].reshape(1, HH)
        beta = p["bn_beta"][i].reshape(1, HH)
        if i < LL - 1:
            h, hw, hroot = _tc_mid(z, stats, gamma, beta, h,
                                   wt[i + 1], p["root"][i + 1],
                                   residual=(i > 0))
        else:
            emb, pq_p, pq_q, na = _tc_last(
                z, stats, gamma, beta, h,
                p["na_W1"], p["na_b1"].reshape(1, HH // 2),
                p["na_W2"], p["na_b2"].reshape(1, 1),
                p["ea_W1"][:HH], p["ea_W1"][HH:2 * HH])

    naw, w1cb = _tc_softmax(na, p["ea_W1"][2 * HH:], p["ea_b1"].reshape(1, HH // 2))
    g = _tc_graph_out(naw, emb, p["W_out"], p["b_out"].reshape(1, OO))

    ea_u = _sc_edge(pq_p, pq_q, srcp, dstp)
    w1cb8 = jnp.pad(w1cb, ((0, 2), (0, 0)))
    ea_full = _tc_ea(ea_u, etp.reshape(E_PAD, 1), w1cb8,
                     p["ea_W2"], p["ea_b2"].reshape(1, 1))
    ea = ea_full[:EE]

    return (g, emb, naw, ea)


# speedup vs baseline: 5.4630x; 1.0001x over previous
import functools

import jax
import jax.numpy as jnp
from jax import lax
from jax.experimental import pallas as pl
from jax.experimental.pallas import tpu as pltpu
from jax.experimental.pallas import tpu_sc as plsc

NN = 10000
EE = 160000
DD = 256
HH = 256
OO = 256
RR = 6
NB = 30
LL = 3

E_PAD = 163840
RN = RR * NN
RN_PAD = 61440
CK = 128
BLK = 400
NBLK = NN // BLK

_f32 = jnp.float32
_i32 = jnp.int32


def _dot(a, b):
    return jnp.dot(a, b, preferred_element_type=_f32)


def _tcw_body(att_ref, basis_ref, out_ref):
    out_ref[...] = _dot(att_ref[0], basis_ref[0])[None]


def _tc_relweights(att, basis):
    bflat = basis.reshape(LL, NB, HH * HH)
    out = pl.pallas_call(
        _tcw_body,
        grid=(LL, (HH * HH) // 2048),
        in_specs=[
            pl.BlockSpec((1, RR, NB), lambda l, c: (l, 0, 0)),
            pl.BlockSpec((1, NB, 2048), lambda l, c: (l, 0, c)),
        ],
        out_specs=pl.BlockSpec((1, RR, 2048), lambda l, c: (l, 0, c)),
        out_shape=jax.ShapeDtypeStruct((LL, RR, HH * HH), _f32),
    )(att, bflat)
    return out.reshape(LL, RR, HH, HH)


def _tcin_body(x_ref, win_ref, bin_ref, w_ref, root_ref,
               h_ref, hw_ref, hroot_ref):
    h = jnp.maximum(_dot(x_ref[...], win_ref[...]) + bin_ref[...], 0.0)
    h_ref[...] = h
    for r in range(RR):
        hw_ref[r] = _dot(h, w_ref[r])
    hroot_ref[...] = _dot(h, root_ref[...])


def _tc_input(x, w_in, b_in, w0, root0):
    return pl.pallas_call(
        _tcin_body,
        grid=(NBLK,),
        in_specs=[
            pl.BlockSpec((BLK, DD), lambda i: (i, 0)),
            pl.BlockSpec((DD, HH), lambda i: (0, 0)),
            pl.BlockSpec((1, HH), lambda i: (0, 0)),
            pl.BlockSpec((RR, HH, HH), lambda i: (0, 0, 0)),
            pl.BlockSpec((HH, HH), lambda i: (0, 0)),
        ],
        out_specs=[
            pl.BlockSpec((BLK, HH), lambda i: (i, 0)),
            pl.BlockSpec((RR, BLK, HH), lambda i: (0, i, 0)),
            pl.BlockSpec((BLK, HH), lambda i: (i, 0)),
        ],
        out_shape=[
            jax.ShapeDtypeStruct((NN, HH), _f32),
            jax.ShapeDtypeStruct((RR, NN, HH), _f32),
            jax.ShapeDtypeStruct((NN, HH), _f32),
        ],
    )(x, w_in, b_in, w0, root0)


def _tcstats_body(hroot_ref, a0_ref, a1_ref, bias_ref, z_ref, st_ref, acc_ref):
    i = pl.program_id(0)

    @pl.when(i == 0)
    def _():
        acc_ref[...] = jnp.zeros_like(acc_ref)

    z = hroot_ref[...] + jnp.concatenate([a0_ref[...], a1_ref[...]], axis=-1) \
        + bias_ref[...]
    z_ref[...] = z
    acc_ref[0:1] += jnp.sum(z, axis=0, keepdims=True)
    acc_ref[1:2] += jnp.sum(z * z, axis=0, keepdims=True)

    @pl.when(i == NBLK - 1)
    def _():
        st_ref[...] = acc_ref[...]


def _tc_stats(hroot, agg0, agg1, bias):
    return pl.pallas_call(
        _tcstats_body,
        grid=(NBLK,),
        in_specs=[
            pl.BlockSpec((BLK, HH), lambda i: (i, 0)),
            pl.BlockSpec((BLK, 128), lambda i: (i, 0)),
            pl.BlockSpec((BLK, 128), lambda i: (i, 0)),
            pl.BlockSpec((1, HH), lambda i: (0, 0)),
        ],
        out_specs=[
            pl.BlockSpec((BLK, HH), lambda i: (i, 0)),
            pl.BlockSpec((2, HH), lambda i: (0, 0)),
        ],
        out_shape=[
            jax.ShapeDtypeStruct((NN, HH), _f32),
            jax.ShapeDtypeStruct((2, HH), _f32),
        ],
        scratch_shapes=[pltpu.VMEM((2, HH), _f32)],
    )(hroot, agg0, agg1, bias)


def _bn_relu(z, st_ref, gamma_ref, beta_ref):
    inv_n = 1.0 / NN
    mu = st_ref[0:1] * inv_n
    var = st_ref[1:2] * inv_n - mu * mu
    rs = lax.rsqrt(var + 1e-5)
    return jnp.maximum((z - mu) * rs * gamma_ref[...] + beta_ref[...], 0.0)


def _tcb_body(z_ref, st_ref, gamma_ref, beta_ref, hprev_ref, w_ref, root_ref,
              h_ref, hw_ref, hroot_ref, *, residual):
    hn = _bn_relu(z_ref[...], st_ref, gamma_ref, beta_ref)
    if residual:
        hn = hn + hprev_ref[...]
    h_ref[...] = hn
    for r in range(RR):
        hw_ref[r] = _dot(hn, w_ref[r])
    hroot_ref[...] = _dot(hn, root_ref[...])


def _tc_mid(z, stats, gamma, beta, hprev, wnext, rootnext, residual):
    return pl.pallas_call(
        functools.partial(_tcb_body, residual=residual),
        grid=(NBLK,),
        in_specs=[
            pl.BlockSpec((BLK, HH), lambda i: (i, 0)),
            pl.BlockSpec((2, HH), lambda i: (0, 0)),
            pl.BlockSpec((1, HH), lambda i: (0, 0)),
            pl.BlockSpec((1, HH), lambda i: (0, 0)),
            pl.BlockSpec((BLK, HH), lambda i: (i, 0)),
            pl.BlockSpec((RR, HH, HH), lambda i: (0, 0, 0)),
            pl.BlockSpec((HH, HH), lambda i: (0, 0)),
        ],
        out_specs=[
            pl.BlockSpec((BLK, HH), lambda i: (i, 0)),
            pl.BlockSpec((RR, BLK, HH), lambda i: (0, i, 0)),
            pl.BlockSpec((BLK, HH), lambda i: (i, 0)),
        ],
        out_shape=[
            jax.ShapeDtypeStruct((NN, HH), _f32),
            jax.ShapeDtypeStruct((RR, NN, HH), _f32),
            jax.ShapeDtypeStruct((NN, HH), _f32),
        ],
    )(z, stats, gamma, beta, hprev, wnext, rootnext)


def _tcb2_body(z_ref, st_ref, gamma_ref, beta_ref, hprev_ref,
               naw1_ref, nab1_ref, naw2_ref, nab2_ref, w1a_ref, w1b_ref,
               emb_ref, p_ref, q_ref, na_ref):
    hn = _bn_relu(z_ref[...], st_ref, gamma_ref, beta_ref) + hprev_ref[...]
    emb_ref[...] = hn
    na1 = jnp.maximum(_dot(hn, naw1_ref[...]) + nab1_ref[...], 0.0)
    na_ref[...] = _dot(na1, naw2_ref[...]) + nab2_ref[...]
    p_ref[...] = _dot(hn, w1a_ref[...])
    q_ref[...] = _dot(hn, w1b_ref[...])


def _tc_last(z, stats, gamma, beta, hprev, na_w1, na_b1, na_w2, na_b2,
             w1a, w1b):
    return pl.pallas_call(
        _tcb2_body,
        grid=(NBLK,),
        in_specs=[
            pl.BlockSpec((BLK, HH), lambda i: (i, 0)),
            pl.BlockSpec((2, HH), lambda i: (0, 0)),
            pl.BlockSpec((1, HH), lambda i: (0, 0)),
            pl.BlockSpec((1, HH), lambda i: (0, 0)),
            pl.BlockSpec((BLK, HH), lambda i: (i, 0)),
            pl.BlockSpec((HH, HH // 2), lambda i: (0, 0)),
            pl.BlockSpec((1, HH // 2), lambda i: (0, 0)),
            pl.BlockSpec((HH // 2, 1), lambda i: (0, 0)),
            pl.BlockSpec((1, 1), lambda i: (0, 0)),
            pl.BlockSpec((HH, HH // 2), lambda i: (0, 0)),
            pl.BlockSpec((HH, HH // 2), lambda i: (0, 0)),
        ],
        out_specs=[
            pl.BlockSpec((BLK, HH), lambda i: (i, 0)),
            pl.BlockSpec((BLK, HH // 2), lambda i: (i, 0)),
            pl.BlockSpec((BLK, HH // 2), lambda i: (i, 0)),
            pl.BlockSpec((BLK, 1), lambda i: (i, 0)),
        ],
        out_shape=[
            jax.ShapeDtypeStruct((NN, HH), _f32),
            jax.ShapeDtypeStruct((NN, HH // 2), _f32),
            jax.ShapeDtypeStruct((NN, HH // 2), _f32),
            jax.ShapeDtypeStruct((NN, 1), _f32),
        ],
    )(z, stats, gamma, beta, hprev, na_w1, na_b1, na_w2, na_b2, w1a, w1b)


def _tcwiden_body(r_ref, o_ref):
    o_ref[...] = jnp.broadcast_to(r_ref[...], (8192, 16))


def _tc_widen(recip):
    return pl.pallas_call(
        _tcwiden_body,
        grid=(E_PAD // 8192,),
        in_specs=[pl.BlockSpec((8192, 1), lambda i: (i, 0))],
        out_specs=pl.BlockSpec((8192, 16), lambda i: (i, 0)),
        out_shape=jax.ShapeDtypeStruct((E_PAD, 16), _f32),
    )(recip.reshape(E_PAD, 1))


def _tcsoft_body(na_ref, w1c_ref, b1_ref, naw_ref, w1cb_ref):
    nav = na_ref[...]
    e = jnp.exp(nav - jnp.max(nav))
    naw_ref[...] = e / jnp.sum(e)
    w1cb_ref[...] = w1c_ref[...] + b1_ref[...]


def _tc_softmax(na, w1c, b1):
    return pl.pallas_call(
        _tcsoft_body,
        out_shape=[
            jax.ShapeDtypeStruct((NN, 1), _f32),
            jax.ShapeDtypeStruct((RR, HH // 2), _f32),
        ],
    )(na, w1c, b1)


def _tcg_body(naw_ref, emb_ref, wout_ref, bout_ref, g_ref, acc_ref):
    i = pl.program_id(0)

    @pl.when(i == 0)
    def _():
        acc_ref[...] = jnp.zeros_like(acc_ref)

    acc_ref[...] += lax.dot_general(
        naw_ref[...], emb_ref[...], (((0,), (0,)), ((), ())),
        preferred_element_type=_f32)

    @pl.when(i == NBLK - 1)
    def _():
        g_ref[...] = _dot(acc_ref[...], wout_ref[...]) + bout_ref[...]


def _tc_graph_out(naw, emb, w_out, b_out):
    return pl.pallas_call(
        _tcg_body,
        grid=(NBLK,),
        in_specs=[
            pl.BlockSpec((BLK, 1), lambda i: (i, 0)),
            pl.BlockSpec((BLK, HH), lambda i: (i, 0)),
            pl.BlockSpec((HH, OO), lambda i: (0, 0)),
            pl.BlockSpec((1, OO), lambda i: (0, 0)),
        ],
        out_specs=pl.BlockSpec((1, OO), lambda i: (0, 0)),
        out_shape=jax.ShapeDtypeStruct((1, OO), _f32),
        scratch_shapes=[pltpu.VMEM((1, OO), _f32)],
    )(naw, emb, w_out, b_out)


_MESH = plsc.VectorSubcoreMesh(core_axis_name="c", subcore_axis_name="s")


def _sc_prep_body(idxrn_hbm, maske_hbm, recip_hbm,
                  cnt_sh, idx_v, val_v, out_v, sem):
    c = lax.axis_index("c")
    s = lax.axis_index("s")

    @pl.when(c == 0)
    def _():
        for k in range(8):
            val_v[pl.ds(k * 16, 16)] = jnp.zeros((16,), _f32)

        @pl.loop(0, 30)
        def _(j):
            pltpu.sync_copy(val_v, cnt_sh.at[pl.ds(s * 3840 + j * 128, 128)])

    plsc.subcore_barrier()

    @pl.when(c == 0)
    def _():
        @pl.loop(0, 80)
        def _(t):
            off = s * 10240 + t * CK
            pltpu.sync_copy(idxrn_hbm.at[pl.ds(off, CK)], idx_v)
            pltpu.sync_copy(maske_hbm.at[pl.ds(off, CK)], val_v)
            pltpu.sync_copy(val_v, cnt_sh.at[idx_v], add=True)

    plsc.subcore_barrier()

    @pl.when(c == 0)
    def _():
        @pl.loop(0, 30)
        def _(j):
            sl = pl.ds(s * 3840 + j * 128, 128)
            pltpu.sync_copy(cnt_sh.at[sl], out_v)
            for k in range(8):
                v = out_v[pl.ds(k * 16, 16)]
                out_v[pl.ds(k * 16, 16)] = 1.0 / jnp.maximum(v, 1.0)
            pltpu.sync_copy(out_v, cnt_sh.at[sl])

    plsc.subcore_barrier()

    @pl.when(c == 0)
    def _():
        @pl.loop(0, 80)
        def _(t):
            off = s * 10240 + t * CK
            pltpu.sync_copy(idxrn_hbm.at[pl.ds(off, CK)], idx_v)
            pltpu.async_copy(cnt_sh.at[idx_v], out_v, sem).wait()
            pltpu.sync_copy(maske_hbm.at[pl.ds(off, CK)], val_v)
            for k in range(8):
                sl = pl.ds(k * 16, 16)
                out_v[sl] = out_v[sl] * val_v[sl]
            pltpu.sync_copy(out_v, recip_hbm.at[pl.ds(off, CK)])


_sc_prep = functools.partial(
    pl.kernel,
    _sc_prep_body,
    out_type=jax.ShapeDtypeStruct((E_PAD,), _f32),
    mesh=_MESH,
    scratch_types=[
        pltpu.VMEM_SHARED((RN_PAD,), _f32),
        pltpu.VMEM((CK,), _i32),
        pltpu.VMEM((CK,), _f32),
        pltpu.VMEM((CK,), _f32),
        pltpu.SemaphoreType.DMA,
    ],
)()


def _sc_agg_body(hw2_hbm, base2_hbm, dst_hbm, recip2_hbm,
                 agg0_hbm, agg1_hbm,
                 acc_sh, rows0_v, rows1_v, bidx0_v, bidx1_v,
                 didx0_v, didx1_v, rrep0_v, rrep1_v, sem0, sem1):
    c = lax.axis_index("c")
    s = lax.axis_index("s")

    @pl.loop(0, CK)
    def _(j):
        for k in range(8):
            rows0_v[j, pl.ds(k * 16, 16)] = jnp.zeros((16,), _f32)

    for o, n in ((0, 128), (128, 128), (256, 128), (384, 128), (512, 120)):
        pltpu.sync_copy(rows0_v.at[pl.ds(0, n)],
                        acc_sh.at[pl.ds(s * 632 + o, n)])

    plsc.subcore_barrier()

    def _issue(off, bidx_v, rows_v, sem):
        pltpu.sync_copy(base2_hbm.at[pl.ds(off, CK)], bidx_v)
        for k in range(8):
            sl = pl.ds(k * 16, 16)
            bidx_v[sl] = bidx_v[sl] + c
        return pltpu.async_copy(hw2_hbm.at[bidx_v], rows_v, sem)

    def _process(off, rows_v, cp):
        pltpu.sync_copy(dst_hbm.at[pl.ds(off, CK)], didx0_v)
        pltpu.sync_copy(recip2_hbm.at[pl.ds(off, CK)], rrep0_v)
        cp.wait()

        @pl.loop(0, CK)
        def _(j):
            rv = rrep0_v[j]
            for k in range(8):
                sl = pl.ds(k * 16, 16)
                rows_v[j, sl] = rows_v[j, sl] * rv

        pltpu.sync_copy(rows_v, acc_sh.at[didx0_v], add=True)

    @pl.loop(0, 40)
    def _(g):
        off0 = s * 10240 + g * (2 * CK)
        off1 = off0 + CK
        cp0 = _issue(off0, bidx0_v, rows0_v, sem0)
        cp1 = _issue(off1, bidx1_v, rows1_v, sem1)
        _process(off0, rows0_v, cp0)
        _process(off1, rows1_v, cp1)

    plsc.subcore_barrier()

    @pl.when(c == 0)
    def _():
        pltpu.sync_copy(acc_sh.at[pl.ds(s * 632, 632)],
                        agg0_hbm.at[pl.ds(s * 632, 632)])

    @pl.when(c == 1)
    def _():
        pltpu.sync_copy(acc_sh.at[pl.ds(s * 632, 632)],
                        agg1_hbm.at[pl.ds(s * 632, 632)])


_sc_agg = functools.partial(
    pl.kernel,
    _sc_agg_body,
    out_type=[
        jax.ShapeDtypeStruct((10112, 128), _f32),
        jax.ShapeDtypeStruct((10112, 128), _f32),
    ],
    mesh=_MESH,
    scratch_types=[
        pltpu.VMEM_SHARED((10112, 128), _f32),
        pltpu.VMEM((CK, 128), _f32),
        pltpu.VMEM((CK, 128), _f32),
        pltpu.VMEM((CK,), _i32),
        pltpu.VMEM((CK,), _i32),
        pltpu.VMEM((CK,), _i32),
        pltpu.VMEM((CK,), _i32),
        pltpu.VMEM((CK, 16), _f32),
        pltpu.VMEM((CK, 16), _f32),
        pltpu.SemaphoreType.DMA,
        pltpu.SemaphoreType.DMA,
    ],
)()


def _sc_edge_body(p_hbm, q_hbm, src_hbm, dst_hbm, u_hbm,
                  prow0_v, prow1_v, qrow0_v, qrow1_v,
                  sidx0_v, sidx1_v, didx0_v, didx1_v, sem0, sem1):
    c = lax.axis_index("c")
    s = lax.axis_index("s")
    wid = s * 2 + c

    def _issue(off, sidx_v, didx_v, prow_v, qrow_v, sem):
        pltpu.sync_copy(src_hbm.at[pl.ds(off, CK)], sidx_v)
        cpp = pltpu.async_copy(p_hbm.at[sidx_v], prow_v, sem)
        pltpu.sync_copy(dst_hbm.at[pl.ds(off, CK)], didx_v)
        cpq = pltpu.async_copy(q_hbm.at[didx_v], qrow_v, sem)
        return cpp, cpq

    def _process(off, prow_v, qrow_v, cps, sem):
        cps[0].wait()
        cps[1].wait()

        @pl.loop(0, 32)
        def _(g):
            for l in range(4):
                j = g * 4 + l
                for k in range(8):
                    sl = pl.ds(k * 16, 16)
                    prow_v[j, sl] = prow_v[j, sl] + qrow_v[j, sl]

        return pltpu.async_copy(prow_v, u_hbm.at[pl.ds(off, CK)], sem)

    @pl.loop(0, 20)
    def _(t):
        off0 = wid * 5120 + t * (2 * CK)
        off1 = off0 + CK
        cp0 = _issue(off0, sidx0_v, didx0_v, prow0_v, qrow0_v, sem0)
        cp1 = _issue(off1, sidx1_v, didx1_v, prow1_v, qrow1_v, sem1)
        cw0 = _process(off0, prow0_v, qrow0_v, cp0, sem0)
        cw1 = _process(off1, prow1_v, qrow1_v, cp1, sem1)
        cw0.wait()
        cw1.wait()


_sc_edge = functools.partial(
    pl.kernel,
    _sc_edge_body,
    out_type=jax.ShapeDtypeStruct((E_PAD, 128), _f32),
    mesh=_MESH,
    scratch_types=[
        pltpu.VMEM((CK, 128), _f32),
        pltpu.VMEM((CK, 128), _f32),
        pltpu.VMEM((CK, 128), _f32),
        pltpu.VMEM((CK, 128), _f32),
        pltpu.VMEM((CK,), _i32),
        pltpu.VMEM((CK,), _i32),
        pltpu.VMEM((CK,), _i32),
        pltpu.VMEM((CK,), _i32),
        pltpu.SemaphoreType.DMA,
        pltpu.SemaphoreType.DMA,
    ],
)()


def _tcea_body(u_ref, et_ref, w1cb_ref, w2_ref, b2_ref, ea_ref):
    oh = (et_ref[...] == lax.broadcasted_iota(_i32, (1, 8), 1)).astype(_f32)
    v = jnp.maximum(u_ref[...] + _dot(oh, w1cb_ref[...]), 0.0)
    v = _dot(v, w2_ref[...]) + b2_ref[...]
    ea_ref[...] = 1.0 / (1.0 + jnp.exp(-v))


def _tc_ea(u, et2, w1cb8, w2, b2):
    return pl.pallas_call(
        _tcea_body,
        grid=(E_PAD // 2048,),
        in_specs=[
            pl.BlockSpec((2048, 128), lambda i: (i, 0)),
            pl.BlockSpec((2048, 1), lambda i: (i, 0)),
            pl.BlockSpec((8, 128), lambda i: (0, 0)),
            pl.BlockSpec((128, 1), lambda i: (0, 0)),
            pl.BlockSpec((1, 1), lambda i: (0, 0)),
        ],
        out_specs=pl.BlockSpec((2048, 1), lambda i: (i, 0)),
        out_shape=jax.ShapeDtypeStruct((E_PAD, 1), _f32),
    )(u, et2, w1cb8, w2, b2)


def kernel(x, edge_index, edge_type, params):
    p = params
    src = edge_index[0].astype(_i32)
    dst = edge_index[1].astype(_i32)
    et = edge_type.astype(_i32)

    pad = E_PAD - EE
    srcp = jnp.pad(src, (0, pad))
    dstp = jnp.pad(dst, (0, pad))
    etp = jnp.pad(et, (0, pad))
    base2 = jnp.pad((et * NN + src) * 2, (0, pad))
    idxrn = jnp.pad(et * NN + dst, (0, pad), constant_values=RN_PAD - 1)
    maske = jnp.pad(jnp.ones((EE,), _f32), (0, pad))

    wt = _tc_relweights(p["att"], p["basis"])

    recip = _sc_prep(idxrn, maske)
    recip2 = _tc_widen(recip)

    h, hw, hroot = _tc_input(x, p["W_in"], p["b_in"].reshape(1, HH),
                             wt[0], p["root"][0])

    emb = naw = na = pq_p = pq_q = None
    for i in range(LL):
        hw2 = hw.reshape(RR * NN * 2, 128)
        agg0, agg1 = _sc_agg(hw2, base2, dstp, recip2)
        z, stats = _tc_stats(hroot, agg0, agg1, p["conv_bias"][i].reshape(1, HH))
        gamma = p["bn_gamma"][i].reshape(1, HH)
        beta = p["bn_beta"][i].reshape(1, HH)
        if i < LL - 1:
            h, hw, hroot = _tc_mid(z, stats, gamma, beta, h,
                                   wt[i + 1], p["root"][i + 1],
                                   residual=(i > 0))
        else:
            emb, pq_p, pq_q, na = _tc_last(
                z, stats, gamma, beta, h,
                p["na_W1"], p["na_b1"].reshape(1, HH // 2),
                p["na_W2"], p["na_b2"].reshape(1, 1),
                p["ea_W1"][:HH], p["ea_W1"][HH:2 * HH])

    naw, w1cb = _tc_softmax(na, p["ea_W1"][2 * HH:], p["ea_b1"].reshape(1, HH // 2))
    g = _tc_graph_out(naw, emb, p["W_out"], p["b_out"].reshape(1, OO))

    ea_u = _sc_edge(pq_p, pq_q, srcp, dstp)
    w1cb8 = jnp.pad(w1cb, ((0, 2), (0, 0)))
    ea_full = _tc_ea(ea_u, etp.reshape(E_PAD, 1), w1cb8,
                     p["ea_W2"], p["ea_b2"].reshape(1, 1))
    ea = ea_full[:EE]

    return (g, emb, naw, ea)


# speedup vs baseline: 5.4667x; 1.0007x over previous
import functools

import jax
import jax.numpy as jnp
from jax import lax
from jax.experimental import pallas as pl
from jax.experimental.pallas import tpu as pltpu
from jax.experimental.pallas import tpu_sc as plsc

NN = 10000
EE = 160000
DD = 256
HH = 256
OO = 256
RR = 6
NB = 30
LL = 3

E_PAD = 163840
RN = RR * NN
RN_PAD = 61440
CK = 128
BLK = 400
NBLK = NN // BLK

_f32 = jnp.float32
_i32 = jnp.int32


def _dot(a, b):
    return jnp.dot(a, b, preferred_element_type=_f32)


def _tcw_body(att_ref, basis_ref, out_ref):
    out_ref[...] = _dot(att_ref[0], basis_ref[0])[None]


def _tc_relweights(att, basis):
    bflat = basis.reshape(LL, NB, HH * HH)
    out = pl.pallas_call(
        _tcw_body,
        grid=(LL, (HH * HH) // 2048),
        in_specs=[
            pl.BlockSpec((1, RR, NB), lambda l, c: (l, 0, 0)),
            pl.BlockSpec((1, NB, 2048), lambda l, c: (l, 0, c)),
        ],
        out_specs=pl.BlockSpec((1, RR, 2048), lambda l, c: (l, 0, c)),
        out_shape=jax.ShapeDtypeStruct((LL, RR, HH * HH), _f32),
    )(att, bflat)
    return out.reshape(LL, RR, HH, HH)


def _tcin_body(x_ref, win_ref, bin_ref, w_ref, root_ref,
               h_ref, hw_ref, hroot_ref):
    h = jnp.maximum(_dot(x_ref[...], win_ref[...]) + bin_ref[...], 0.0)
    h_ref[...] = h
    for r in range(RR):
        hw_ref[r] = _dot(h, w_ref[r])
    hroot_ref[...] = _dot(h, root_ref[...])


def _tc_input(x, w_in, b_in, w0, root0):
    return pl.pallas_call(
        _tcin_body,
        grid=(NBLK,),
        in_specs=[
            pl.BlockSpec((BLK, DD), lambda i: (i, 0)),
            pl.BlockSpec((DD, HH), lambda i: (0, 0)),
            pl.BlockSpec((1, HH), lambda i: (0, 0)),
            pl.BlockSpec((RR, HH, HH), lambda i: (0, 0, 0)),
            pl.BlockSpec((HH, HH), lambda i: (0, 0)),
        ],
        out_specs=[
            pl.BlockSpec((BLK, HH), lambda i: (i, 0)),
            pl.BlockSpec((RR, BLK, HH), lambda i: (0, i, 0)),
            pl.BlockSpec((BLK, HH), lambda i: (i, 0)),
        ],
        out_shape=[
            jax.ShapeDtypeStruct((NN, HH), _f32),
            jax.ShapeDtypeStruct((RR, NN, HH), _f32),
            jax.ShapeDtypeStruct((NN, HH), _f32),
        ],
    )(x, w_in, b_in, w0, root0)


def _tcstats_body(hroot_ref, a0_ref, a1_ref, bias_ref, z_ref, st_ref, acc_ref):
    i = pl.program_id(0)

    @pl.when(i == 0)
    def _():
        acc_ref[...] = jnp.zeros_like(acc_ref)

    z = hroot_ref[...] + jnp.concatenate([a0_ref[...], a1_ref[...]], axis=-1) \
        + bias_ref[...]
    z_ref[...] = z
    acc_ref[0:1] += jnp.sum(z, axis=0, keepdims=True)
    acc_ref[1:2] += jnp.sum(z * z, axis=0, keepdims=True)

    @pl.when(i == NBLK - 1)
    def _():
        st_ref[...] = acc_ref[...]


def _tc_stats(hroot, agg0, agg1, bias):
    return pl.pallas_call(
        _tcstats_body,
        grid=(NBLK,),
        in_specs=[
            pl.BlockSpec((BLK, HH), lambda i: (i, 0)),
            pl.BlockSpec((BLK, 128), lambda i: (i, 0)),
            pl.BlockSpec((BLK, 128), lambda i: (i, 0)),
            pl.BlockSpec((1, HH), lambda i: (0, 0)),
        ],
        out_specs=[
            pl.BlockSpec((BLK, HH), lambda i: (i, 0)),
            pl.BlockSpec((2, HH), lambda i: (0, 0)),
        ],
        out_shape=[
            jax.ShapeDtypeStruct((NN, HH), _f32),
            jax.ShapeDtypeStruct((2, HH), _f32),
        ],
        scratch_shapes=[pltpu.VMEM((2, HH), _f32)],
    )(hroot, agg0, agg1, bias)


def _bn_relu(z, st_ref, gamma_ref, beta_ref):
    inv_n = 1.0 / NN
    mu = st_ref[0:1] * inv_n
    var = st_ref[1:2] * inv_n - mu * mu
    rs = lax.rsqrt(var + 1e-5)
    return jnp.maximum((z - mu) * rs * gamma_ref[...] + beta_ref[...], 0.0)


def _tcb_body(z_ref, st_ref, gamma_ref, beta_ref, hprev_ref, w_ref, root_ref,
              h_ref, hw_ref, hroot_ref, *, residual):
    hn = _bn_relu(z_ref[...], st_ref, gamma_ref, beta_ref)
    if residual:
        hn = hn + hprev_ref[...]
    h_ref[...] = hn
    for r in range(RR):
        hw_ref[r] = _dot(hn, w_ref[r])
    hroot_ref[...] = _dot(hn, root_ref[...])


def _tc_mid(z, stats, gamma, beta, hprev, wnext, rootnext, residual):
    return pl.pallas_call(
        functools.partial(_tcb_body, residual=residual),
        grid=(NBLK,),
        in_specs=[
            pl.BlockSpec((BLK, HH), lambda i: (i, 0)),
            pl.BlockSpec((2, HH), lambda i: (0, 0)),
            pl.BlockSpec((1, HH), lambda i: (0, 0)),
            pl.BlockSpec((1, HH), lambda i: (0, 0)),
            pl.BlockSpec((BLK, HH), lambda i: (i, 0)),
            pl.BlockSpec((RR, HH, HH), lambda i: (0, 0, 0)),
            pl.BlockSpec((HH, HH), lambda i: (0, 0)),
        ],
        out_specs=[
            pl.BlockSpec((BLK, HH), lambda i: (i, 0)),
            pl.BlockSpec((RR, BLK, HH), lambda i: (0, i, 0)),
            pl.BlockSpec((BLK, HH), lambda i: (i, 0)),
        ],
        out_shape=[
            jax.ShapeDtypeStruct((NN, HH), _f32),
            jax.ShapeDtypeStruct((RR, NN, HH), _f32),
            jax.ShapeDtypeStruct((NN, HH), _f32),
        ],
    )(z, stats, gamma, beta, hprev, wnext, rootnext)


def _tcb2_body(z_ref, st_ref, gamma_ref, beta_ref, hprev_ref,
               naw1_ref, nab1_ref, naw2_ref, nab2_ref, w1a_ref, w1b_ref,
               emb_ref, p_ref, q_ref, na_ref):
    hn = _bn_relu(z_ref[...], st_ref, gamma_ref, beta_ref) + hprev_ref[...]
    emb_ref[...] = hn
    na1 = jnp.maximum(_dot(hn, naw1_ref[...]) + nab1_ref[...], 0.0)
    na_ref[...] = _dot(na1, naw2_ref[...]) + nab2_ref[...]
    p_ref[...] = _dot(hn, w1a_ref[...])
    q_ref[...] = _dot(hn, w1b_ref[...])


def _tc_last(z, stats, gamma, beta, hprev, na_w1, na_b1, na_w2, na_b2,
             w1a, w1b):
    return pl.pallas_call(
        _tcb2_body,
        grid=(NBLK,),
        in_specs=[
            pl.BlockSpec((BLK, HH), lambda i: (i, 0)),
            pl.BlockSpec((2, HH), lambda i: (0, 0)),
            pl.BlockSpec((1, HH), lambda i: (0, 0)),
            pl.BlockSpec((1, HH), lambda i: (0, 0)),
            pl.BlockSpec((BLK, HH), lambda i: (i, 0)),
            pl.BlockSpec((HH, HH // 2), lambda i: (0, 0)),
            pl.BlockSpec((1, HH // 2), lambda i: (0, 0)),
            pl.BlockSpec((HH // 2, 1), lambda i: (0, 0)),
            pl.BlockSpec((1, 1), lambda i: (0, 0)),
            pl.BlockSpec((HH, HH // 2), lambda i: (0, 0)),
            pl.BlockSpec((HH, HH // 2), lambda i: (0, 0)),
        ],
        out_specs=[
            pl.BlockSpec((BLK, HH), lambda i: (i, 0)),
            pl.BlockSpec((BLK, HH // 2), lambda i: (i, 0)),
            pl.BlockSpec((BLK, HH // 2), lambda i: (i, 0)),
            pl.BlockSpec((BLK, 1), lambda i: (i, 0)),
        ],
        out_shape=[
            jax.ShapeDtypeStruct((NN, HH), _f32),
            jax.ShapeDtypeStruct((NN, HH // 2), _f32),
            jax.ShapeDtypeStruct((NN, HH // 2), _f32),
            jax.ShapeDtypeStruct((NN, 1), _f32),
        ],
    )(z, stats, gamma, beta, hprev, na_w1, na_b1, na_w2, na_b2, w1a, w1b)


def _tcwiden_body(r_ref, o_ref):
    o_ref[...] = jnp.broadcast_to(r_ref[...], (8192, 16))


def _tc_widen(recip):
    return pl.pallas_call(
        _tcwiden_body,
        grid=(E_PAD // 8192,),
        in_specs=[pl.BlockSpec((8192, 1), lambda i: (i, 0))],
        out_specs=pl.BlockSpec((8192, 16), lambda i: (i, 0)),
        out_shape=jax.ShapeDtypeStruct((E_PAD, 16), _f32),
    )(recip.reshape(E_PAD, 1))


def _tcsoft_body(na_ref, w1c_ref, b1_ref, naw_ref, w1cb_ref):
    nav = na_ref[...]
    e = jnp.exp(nav - jnp.max(nav))
    naw_ref[...] = e / jnp.sum(e)
    w1cb_ref[...] = w1c_ref[...] + b1_ref[...]


def _tc_softmax(na, w1c, b1):
    return pl.pallas_call(
        _tcsoft_body,
        out_shape=[
            jax.ShapeDtypeStruct((NN, 1), _f32),
            jax.ShapeDtypeStruct((RR, HH // 2), _f32),
        ],
    )(na, w1c, b1)


def _tcg_body(naw_ref, emb_ref, wout_ref, bout_ref, g_ref, acc_ref):
    i = pl.program_id(0)

    @pl.when(i == 0)
    def _():
        acc_ref[...] = jnp.zeros_like(acc_ref)

    acc_ref[...] += lax.dot_general(
        naw_ref[...], emb_ref[...], (((0,), (0,)), ((), ())),
        preferred_element_type=_f32)

    @pl.when(i == NBLK - 1)
    def _():
        g_ref[...] = _dot(acc_ref[...], wout_ref[...]) + bout_ref[...]


def _tc_graph_out(naw, emb, w_out, b_out):
    return pl.pallas_call(
        _tcg_body,
        grid=(NBLK,),
        in_specs=[
            pl.BlockSpec((BLK, 1), lambda i: (i, 0)),
            pl.BlockSpec((BLK, HH), lambda i: (i, 0)),
            pl.BlockSpec((HH, OO), lambda i: (0, 0)),
            pl.BlockSpec((1, OO), lambda i: (0, 0)),
        ],
        out_specs=pl.BlockSpec((1, OO), lambda i: (0, 0)),
        out_shape=jax.ShapeDtypeStruct((1, OO), _f32),
        scratch_shapes=[pltpu.VMEM((1, OO), _f32)],
    )(naw, emb, w_out, b_out)


_MESH = plsc.VectorSubcoreMesh(core_axis_name="c", subcore_axis_name="s")


def _sc_prep_body(idxrn_hbm, maske_hbm, recip_hbm,
                  cnt_sh, idx_v, val_v, out_v, sem):
    c = lax.axis_index("c")
    s = lax.axis_index("s")

    @pl.when(c == 0)
    def _():
        for k in range(8):
            val_v[pl.ds(k * 16, 16)] = jnp.zeros((16,), _f32)

        @pl.loop(0, 30)
        def _(j):
            pltpu.sync_copy(val_v, cnt_sh.at[pl.ds(s * 3840 + j * 128, 128)])

    plsc.subcore_barrier()

    @pl.when(c == 0)
    def _():
        @pl.loop(0, 80)
        def _(t):
            off = s * 10240 + t * CK
            pltpu.sync_copy(idxrn_hbm.at[pl.ds(off, CK)], idx_v)
            pltpu.sync_copy(maske_hbm.at[pl.ds(off, CK)], val_v)
            pltpu.sync_copy(val_v, cnt_sh.at[idx_v], add=True)

    plsc.subcore_barrier()

    @pl.when(c == 0)
    def _():
        @pl.loop(0, 30)
        def _(j):
            sl = pl.ds(s * 3840 + j * 128, 128)
            pltpu.sync_copy(cnt_sh.at[sl], out_v)
            for k in range(8):
                v = out_v[pl.ds(k * 16, 16)]
                out_v[pl.ds(k * 16, 16)] = 1.0 / jnp.maximum(v, 1.0)
            pltpu.sync_copy(out_v, cnt_sh.at[sl])

    plsc.subcore_barrier()

    @pl.when(c == 0)
    def _():
        @pl.loop(0, 80)
        def _(t):
            off = s * 10240 + t * CK
            pltpu.sync_copy(idxrn_hbm.at[pl.ds(off, CK)], idx_v)
            pltpu.async_copy(cnt_sh.at[idx_v], out_v, sem).wait()
            pltpu.sync_copy(maske_hbm.at[pl.ds(off, CK)], val_v)
            for k in range(8):
                sl = pl.ds(k * 16, 16)
                out_v[sl] = out_v[sl] * val_v[sl]
            pltpu.sync_copy(out_v, recip_hbm.at[pl.ds(off, CK)])


_sc_prep = functools.partial(
    pl.kernel,
    _sc_prep_body,
    out_type=jax.ShapeDtypeStruct((E_PAD,), _f32),
    mesh=_MESH,
    scratch_types=[
        pltpu.VMEM_SHARED((RN_PAD,), _f32),
        pltpu.VMEM((CK,), _i32),
        pltpu.VMEM((CK,), _f32),
        pltpu.VMEM((CK,), _f32),
        pltpu.SemaphoreType.DMA,
    ],
)()


def _sc_agg_body(hw2_hbm, base2_hbm, dst_hbm, recip2_hbm,
                 agg0_hbm, agg1_hbm,
                 acc_sh, rows0_v, rows1_v, bidx0_v, bidx1_v,
                 didx0_v, rrep0_v, sem0, sem1):
    c = lax.axis_index("c")
    s = lax.axis_index("s")

    @pl.loop(0, CK)
    def _(j):
        for k in range(8):
            rows0_v[j, pl.ds(k * 16, 16)] = jnp.zeros((16,), _f32)

    for o, n in ((0, 128), (128, 128), (256, 128), (384, 128), (512, 120)):
        pltpu.sync_copy(rows0_v.at[pl.ds(0, n)],
                        acc_sh.at[pl.ds(s * 632 + o, n)])

    plsc.subcore_barrier()

    def _issue(off, bidx_v, rows_v, sem):
        pltpu.sync_copy(base2_hbm.at[pl.ds(off, CK)], bidx_v)
        for k in range(8):
            sl = pl.ds(k * 16, 16)
            bidx_v[sl] = bidx_v[sl] + c
        return pltpu.async_copy(hw2_hbm.at[bidx_v], rows_v, sem)

    def _process(off, rows_v, cp):
        pltpu.sync_copy(dst_hbm.at[pl.ds(off, CK)], didx0_v)
        pltpu.sync_copy(recip2_hbm.at[pl.ds(off, CK)], rrep0_v)
        cp.wait()

        @pl.loop(0, CK)
        def _(j):
            rv = rrep0_v[j]
            for k in range(8):
                sl = pl.ds(k * 16, 16)
                rows_v[j, sl] = rows_v[j, sl] * rv

        pltpu.sync_copy(rows_v, acc_sh.at[didx0_v], add=True)

    @pl.loop(0, 40)
    def _(g):
        off0 = s * 10240 + g * (2 * CK)
        off1 = off0 + CK
        cp0 = _issue(off0, bidx0_v, rows0_v, sem0)
        cp1 = _issue(off1, bidx1_v, rows1_v, sem1)
        _process(off0, rows0_v, cp0)
        _process(off1, rows1_v, cp1)

    plsc.subcore_barrier()

    @pl.when(c == 0)
    def _():
        pltpu.sync_copy(acc_sh.at[pl.ds(s * 632, 632)],
                        agg0_hbm.at[pl.ds(s * 632, 632)])

    @pl.when(c == 1)
    def _():
        pltpu.sync_copy(acc_sh.at[pl.ds(s * 632, 632)],
                        agg1_hbm.at[pl.ds(s * 632, 632)])


_sc_agg = functools.partial(
    pl.kernel,
    _sc_agg_body,
    out_type=[
        jax.ShapeDtypeStruct((10112, 128), _f32),
        jax.ShapeDtypeStruct((10112, 128), _f32),
    ],
    mesh=_MESH,
    scratch_types=[
        pltpu.VMEM_SHARED((10112, 128), _f32),
        pltpu.VMEM((CK, 128), _f32),
        pltpu.VMEM((CK, 128), _f32),
        pltpu.VMEM((CK,), _i32),
        pltpu.VMEM((CK,), _i32),
        pltpu.VMEM((CK,), _i32),
        pltpu.VMEM((CK, 16), _f32),
        pltpu.SemaphoreType.DMA,
        pltpu.SemaphoreType.DMA,
    ],
)()


def _sc_edge_body(p_hbm, q_hbm, src_hbm, dst_hbm, u_hbm,
                  prow0_v, prow1_v, qrow0_v, qrow1_v,
                  sidx0_v, sidx1_v, didx0_v, didx1_v, sem0, sem1):
    c = lax.axis_index("c")
    s = lax.axis_index("s")
    wid = s * 2 + c

    def _issue(off, sidx_v, didx_v, prow_v, qrow_v, sem):
        pltpu.sync_copy(src_hbm.at[pl.ds(off, CK)], sidx_v)
        cpp = pltpu.async_copy(p_hbm.at[sidx_v], prow_v, sem)
        pltpu.sync_copy(dst_hbm.at[pl.ds(off, CK)], didx_v)
        cpq = pltpu.async_copy(q_hbm.at[didx_v], qrow_v, sem)
        return cpp, cpq

    def _process(off, prow_v, qrow_v, cps, sem):
        cps[0].wait()
        cps[1].wait()

        @pl.loop(0, 32)
        def _(g):
            for l in range(4):
                j = g * 4 + l
                for k in range(8):
                    sl = pl.ds(k * 16, 16)
                    prow_v[j, sl] = prow_v[j, sl] + qrow_v[j, sl]

        return pltpu.async_copy(prow_v, u_hbm.at[pl.ds(off, CK)], sem)

    @pl.loop(0, 20)
    def _(t):
        off0 = wid * 5120 + t * (2 * CK)
        off1 = off0 + CK
        cp0 = _issue(off0, sidx0_v, didx0_v, prow0_v, qrow0_v, sem0)
        cp1 = _issue(off1, sidx1_v, didx1_v, prow1_v, qrow1_v, sem1)
        cw0 = _process(off0, prow0_v, qrow0_v, cp0, sem0)
        cw1 = _process(off1, prow1_v, qrow1_v, cp1, sem1)
        cw0.wait()
        cw1.wait()


_sc_edge = functools.partial(
    pl.kernel,
    _sc_edge_body,
    out_type=jax.ShapeDtypeStruct((E_PAD, 128), _f32),
    mesh=_MESH,
    scratch_types=[
        pltpu.VMEM((CK, 128), _f32),
        pltpu.VMEM((CK, 128), _f32),
        pltpu.VMEM((CK, 128), _f32),
        pltpu.VMEM((CK, 128), _f32),
        pltpu.VMEM((CK,), _i32),
        pltpu.VMEM((CK,), _i32),
        pltpu.VMEM((CK,), _i32),
        pltpu.VMEM((CK,), _i32),
        pltpu.SemaphoreType.DMA,
        pltpu.SemaphoreType.DMA,
    ],
)()


def _tcea_body(u_ref, et_ref, w1cb_ref, w2_ref, b2_ref, ea_ref):
    oh = (et_ref[...] == lax.broadcasted_iota(_i32, (1, 8), 1)).astype(_f32)
    v = jnp.maximum(u_ref[...] + _dot(oh, w1cb_ref[...]), 0.0)
    v = _dot(v, w2_ref[...]) + b2_ref[...]
    ea_ref[...] = 1.0 / (1.0 + jnp.exp(-v))


def _tc_ea(u, et2, w1cb8, w2, b2):
    return pl.pallas_call(
        _tcea_body,
        grid=(E_PAD // 2048,),
        in_specs=[
            pl.BlockSpec((2048, 128), lambda i: (i, 0)),
            pl.BlockSpec((2048, 1), lambda i: (i, 0)),
            pl.BlockSpec((8, 128), lambda i: (0, 0)),
            pl.BlockSpec((128, 1), lambda i: (0, 0)),
            pl.BlockSpec((1, 1), lambda i: (0, 0)),
        ],
        out_specs=pl.BlockSpec((2048, 1), lambda i: (i, 0)),
        out_shape=jax.ShapeDtypeStruct((E_PAD, 1), _f32),
    )(u, et2, w1cb8, w2, b2)


def kernel(x, edge_index, edge_type, params):
    p = params
    src = edge_index[0].astype(_i32)
    dst = edge_index[1].astype(_i32)
    et = edge_type.astype(_i32)

    pad = E_PAD - EE
    srcp = jnp.pad(src, (0, pad))
    dstp = jnp.pad(dst, (0, pad))
    etp = jnp.pad(et, (0, pad))
    base2 = jnp.pad((et * NN + src) * 2, (0, pad))
    idxrn = jnp.pad(et * NN + dst, (0, pad), constant_values=RN_PAD - 1)
    maske = jnp.pad(jnp.ones((EE,), _f32), (0, pad))

    wt = _tc_relweights(p["att"], p["basis"])

    recip = _sc_prep(idxrn, maske)
    recip2 = _tc_widen(recip)

    h, hw, hroot = _tc_input(x, p["W_in"], p["b_in"].reshape(1, HH),
                             wt[0], p["root"][0])

    emb = naw = na = pq_p = pq_q = None
    for i in range(LL):
        hw2 = hw.reshape(RR * NN * 2, 128)
        agg0, agg1 = _sc_agg(hw2, base2, dstp, recip2)
        z, stats = _tc_stats(hroot, agg0, agg1, p["conv_bias"][i].reshape(1, HH))
        gamma = p["bn_gamma"][i].reshape(1, HH)
        beta = p["bn_beta"][i].reshape(1, HH)
        if i < LL - 1:
            h, hw, hroot = _tc_mid(z, stats, gamma, beta, h,
                                   wt[i + 1], p["root"][i + 1],
                                   residual=(i > 0))
        else:
            emb, pq_p, pq_q, na = _tc_last(
                z, stats, gamma, beta, h,
                p["na_W1"], p["na_b1"].reshape(1, HH // 2),
                p["na_W2"], p["na_b2"].reshape(1, 1),
                p["ea_W1"][:HH], p["ea_W1"][HH:2 * HH])

    naw, w1cb = _tc_softmax(na, p["ea_W1"][2 * HH:], p["ea_b1"].reshape(1, HH // 2))
    g = _tc_graph_out(naw, emb, p["W_out"], p["b_out"].reshape(1, OO))

    ea_u = _sc_edge(pq_p, pq_q, srcp, dstp)
    w1cb8 = jnp.pad(w1cb, ((0, 2), (0, 0)))
    ea_full = _tc_ea(ea_u, etp.reshape(E_PAD, 1), w1cb8,
                     p["ea_W2"], p["ea_b2"].reshape(1, 1))
    ea = ea_full[:EE]

    return (g, emb, naw, ea)
